# Initial kernel scaffold; baseline (speedup 1.0000x reference)
import jax
import jax.numpy as jnp
from jax.experimental import pallas as pl


def kernel(image, depth, T, R, K, Kinv):
    raise NotImplementedError("write your pallas kernel here")


# speedup vs baseline: 27.0085x; 27.0085x over previous
import jax
import jax.numpy as jnp
from jax import lax
from jax.experimental import pallas as pl
from jax.experimental.pallas import tpu as pltpu
from jax.experimental.pallas import tpu_sc as plsc

def _take16(x, idx):
    dnums = lax.GatherDimensionNumbers(
        offset_dims=(), collapsed_slice_dims=(0,), start_index_map=(0,))
    return lax.gather(x, idx[:, None], dnums, slice_sizes=(1,),
                      mode=lax.GatherScatterMode.PROMISE_IN_BOUNDS)


B, C, H, W = 16, 3, 512, 512
HW = H * W
MIN_DEPTH = 0.1

NSEC = 4
SECSZ = HW // NSEC
WIN = 2048
NWIN = HW // WIN
CHUNK = 4096
LANES = 16
NCORES = 2
NSUB = 16


def _rnd(v):
    return v.astype(jnp.bfloat16).astype(jnp.float32)


def _cells_body(m_ref, k_ref, t_ref, depth_ref, cell_ref):
    b = pl.program_id(0)
    gx = _rnd(lax.broadcasted_iota(jnp.int32, (H, W), 0).astype(jnp.float32))
    gy = _rnd(lax.broadcasted_iota(jnp.int32, (H, W), 1).astype(jnp.float32))
    d = jnp.clip(depth_ref[0], MIN_DEPTH)
    t0 = _rnd(t_ref[b, 0] / d)
    t1 = _rnd(t_ref[b, 1] / d)
    t2 = _rnd(t_ref[b, 2] / d)

    def proj_row(i):
        x = (m_ref[b, i, 0] * gx + m_ref[b, i, 1] * gy) + m_ref[b, i, 2]
        y = (k_ref[i, 0] * t0 + k_ref[i, 1] * t1) + k_ref[i, 2] * t2
        ti = x + y
        return jnp.where(ti == 0.0, 0.0001, ti)

    n0 = proj_row(0)
    n1 = proj_row(1)
    n2 = proj_row(2)
    p0 = jnp.clip(n0 / n2, 0.0, float(H - 1)).astype(jnp.int32)
    p1 = jnp.clip(n1 / n2, 0.0, float(W - 1)).astype(jnp.int32)
    cell = p0 * W + p1
    nxt_col = jnp.roll(cell, -1, axis=1)
    nxt_row = jnp.roll(jnp.roll(cell, -1, axis=0), -1, axis=1)
    col = lax.broadcasted_iota(jnp.int32, (H, W), 1)
    row = lax.broadcasted_iota(jnp.int32, (H, W), 0)
    nxt = jnp.where(col < W - 1, nxt_col, nxt_row)
    keep = (cell != nxt) | ((row == H - 1) & (col == W - 1))
    cell_ref[0] = jnp.where(keep, cell, -1)


def _compute_cells(depth, M, K, T2):
    return pl.pallas_call(
        _cells_body,
        grid=(B,),
        in_specs=[
            pl.BlockSpec(memory_space=pltpu.SMEM),
            pl.BlockSpec(memory_space=pltpu.SMEM),
            pl.BlockSpec(memory_space=pltpu.SMEM),
            pl.BlockSpec((1, H, W), lambda b: (b, 0, 0)),
        ],
        out_specs=pl.BlockSpec((1, H, W), lambda b: (b, 0, 0)),
        out_shape=jax.ShapeDtypeStruct((B, H, W), jnp.int32),
    )(M, K, T2, depth)


def _winner_body(cell_hbm, winner_hbm, cellwin, winner_v, sem0, sem1):
    cid = lax.axis_index("c")
    sid = lax.axis_index("s")
    wid = sid * NCORES + cid
    batch = wid // 2
    half = wid % 2
    lane = lax.iota(jnp.int32, 16)
    nxt = jnp.minimum(lane + 1, 15)
    base0 = batch * HW
    sems = (sem0, sem1)

    for sp in range(2):
        sec = half * 2 + sp

        def zero_body(i, _):
            winner_v[pl.ds(i * 16, 16)] = jnp.zeros((16,), jnp.int32)
            return 0

        lax.fori_loop(0, SECSZ // 16, zero_body, 0)

        def process(bi, w):
            def vec_body(v, _):
                cells = cellwin[pl.ds(bi * WIN + v * 16, 16)]
                in_sec = lax.shift_right_logical(cells, 16) == sec
                anyv = jnp.max(plsc.all_reduce_population_count(in_sec)) > 0

                def do(_):
                    local = cells & 0xFFFF
                    key = jnp.where(in_sec, (local << 4) | lane,
                                    jnp.int32(0x40000000) | lane)
                    kp1 = (w * WIN + v * 16 + 1) + lane
                    skey, sval = plsc.sort_key_val(key, kp1)
                    sidx = lax.shift_right_logical(skey, 4)
                    nkey = _take16(skey, nxt)
                    is_last = (sidx != lax.shift_right_logical(nkey, 4)) | (lane == 15)
                    m = is_last & (skey < (1 << 20))
                    plsc.store_scatter(winner_v, [sidx], sval, mask=m)
                    return 0

                lax.cond(anyv, do, lambda _: 0, 0)
                return 0

            lax.fori_loop(0, WIN // 16, vec_body, 0)

        def start(bi, w):
            pltpu.async_copy(cell_hbm.at[pl.ds(base0 + w * WIN, WIN)],
                             cellwin.at[pl.ds(bi * WIN, WIN)], sems[bi])

        def wait(bi, w):
            pltpu.make_async_copy(cell_hbm.at[pl.ds(base0 + w * WIN, WIN)],
                                  cellwin.at[pl.ds(bi * WIN, WIN)],
                                  sems[bi]).wait()

        start(0, jnp.int32(0))

        def pair_body(p, _):
            w0 = 2 * p
            start(1, w0 + 1)
            wait(0, w0)
            process(0, w0)

            @pl.when(p + 1 < NWIN // 2)
            def _():
                start(0, w0 + 2)

            wait(1, w0 + 1)
            process(1, w0 + 1)
            return 0

        lax.fori_loop(0, NWIN // 2, pair_body, 0)
        pltpu.sync_copy(winner_v,
                        winner_hbm.at[pl.ds(batch * HW + sec * SECSZ, SECSZ)])


def _winner_scatter(cells_flat):
    return pl.kernel(
        _winner_body,
        out_type=jax.ShapeDtypeStruct((B * HW,), jnp.int32),
        mesh=plsc.VectorSubcoreMesh(core_axis_name="c", subcore_axis_name="s",
                                    num_cores=NCORES, num_subcores=NSUB),
        scratch_types=[
            pltpu.VMEM((2 * WIN,), jnp.int32),
            pltpu.VMEM((SECSZ,), jnp.int32),
            pltpu.SemaphoreType.DMA,
            pltpu.SemaphoreType.DMA,
        ],
        compiler_params=pltpu.CompilerParams(needs_layout_passes=False),
    )(cells_flat)


def _value_body(winner_hbm, image_hbm, out_hbm, win_v, idxb, posb, valb, sem):
    cid = lax.axis_index("c")
    sid = lax.axis_index("s")
    wid = sid * NCORES + cid
    lane = lax.iota(jnp.int32, 16)
    swap = lane ^ 1

    def fire(rem):
        pltpu.async_copy(image_hbm.at[idxb.at[pl.ds(0, CHUNK)]], valb, sem).wait()

        def scat(j, _):
            vals = valb[pl.ds(j * 16, 16)]
            pos = posb[pl.ds(j * 16, 16)]
            msk = (j * 16 + lane) < rem
            plsc.store_scatter(win_v, [pos], plsc.bitcast(vals, jnp.int32),
                               mask=msk)
            return 0

        lax.fori_loop(0, CHUNK // 16, scat, 0)

    for tsk in range(2):
        t = wid * 2 + tsk
        batch = t // NSEC
        sec = t % NSEC
        for c in range(C):
            base_img = (batch * C + c) * HW

            pltpu.sync_copy(
                winner_hbm.at[pl.ds(batch * HW + sec * SECSZ, SECSZ)], win_v)

            def init_body(i, _):
                idxb[pl.ds(i * 16, 16)] = jnp.broadcast_to(base_img, (16,)).astype(jnp.int32)
                return 0

            lax.fori_loop(0, (CHUNK + 16) // 16, init_body, 0)

            def compact_body(v, off):
                wv = win_v[pl.ds(v * 16, 16)]
                m = wv > 0
                cnt = jnp.sum(m.astype(jnp.int32))

                def active(off):
                    plsc.store_compressed(idxb.at[pl.ds(off, 16)],
                                          (wv - 1) + base_img, mask=m)
                    plsc.store_compressed(posb.at[pl.ds(off, 16)],
                                          v * 16 + lane, mask=m)
                    off = off + cnt

                    def do_fire(off):
                        fire(jnp.int32(CHUNK))
                        idxb[pl.ds(0, 16)] = idxb[pl.ds(CHUNK, 16)]
                        posb[pl.ds(0, 16)] = posb[pl.ds(CHUNK, 16)]
                        return off - CHUNK

                    return lax.cond(off >= CHUNK, do_fire, lambda o: o, off)

                return lax.cond(cnt > 0, active, lambda o: o, off)

            off_fin = lax.fori_loop(0, SECSZ // 16, compact_body,
                                    jnp.int32(0))

            def tail_fire(off):
                fire(off)
                return 0

            lax.cond(off_fin > 0, tail_fire, lambda o: 0, off_fin)

            def pool_body(r, _):
                def grp_body(g, _):
                    a = plsc.bitcast(win_v[pl.ds((2 * r) * W + g * 16, 16)],
                                     jnp.float32)
                    b = plsc.bitcast(win_v[pl.ds((2 * r + 1) * W + g * 16, 16)],
                                     jnp.float32)
                    v = jnp.maximum(a, b)
                    sw = _take16(v, swap)
                    wz = plsc.bitcast(jnp.maximum(v, sw), jnp.int32)
                    win_v[pl.ds((2 * r) * W + g * 16, 16)] = wz
                    win_v[pl.ds((2 * r + 1) * W + g * 16, 16)] = wz
                    return 0

                lax.fori_loop(0, W // 16, grp_body, 0)
                return 0

            lax.fori_loop(0, SECSZ // (2 * W), pool_body, 0)

            out_off = ((batch * C + c) * H + sec * (H // NSEC)) * W
            pltpu.sync_copy(win_v, out_hbm.at[pl.ds(out_off, SECSZ)])


def _gather_pool(winner, image_flat):
    return pl.kernel(
        _value_body,
        out_type=jax.ShapeDtypeStruct((B * C * HW,), jnp.int32),
        mesh=plsc.VectorSubcoreMesh(core_axis_name="c", subcore_axis_name="s",
                                    num_cores=NCORES, num_subcores=NSUB),
        scratch_types=[
            pltpu.VMEM((SECSZ,), jnp.int32),
            pltpu.VMEM((CHUNK + 16,), jnp.int32),
            pltpu.VMEM((CHUNK + 16,), jnp.int32),
            pltpu.VMEM((CHUNK,), jnp.float32),
            pltpu.SemaphoreType.DMA,
        ],
        compiler_params=pltpu.CompilerParams(needs_layout_passes=False),
    )(winner, image_flat)


def kernel(image, depth, T, R, K, Kinv):
    M = jnp.matmul(jnp.matmul(K, R), Kinv)
    Mr = M.astype(jnp.bfloat16).astype(jnp.float32)
    Kr = K.astype(jnp.bfloat16).astype(jnp.float32)
    T2 = T[:, :, 0]
    cells = _compute_cells(depth, Mr, Kr, T2)
    winner = _winner_scatter(cells.reshape(B * HW))
    out_i = _gather_pool(winner, image.reshape(B * C * HW))
    return lax.bitcast_convert_type(out_i, jnp.float32).reshape(B, C, H, W)


# speedup vs baseline: 40.0669x; 1.4835x over previous
import jax
import jax.numpy as jnp
from jax import lax
from jax.experimental import pallas as pl
from jax.experimental.pallas import tpu as pltpu
from jax.experimental.pallas import tpu_sc as plsc

def _take16(x, idx):
    dnums = lax.GatherDimensionNumbers(
        offset_dims=(), collapsed_slice_dims=(0,), start_index_map=(0,))
    return lax.gather(x, idx[:, None], dnums, slice_sizes=(1,),
                      mode=lax.GatherScatterMode.PROMISE_IN_BOUNDS)


B, C, H, W = 16, 3, 512, 512
HW = H * W
MIN_DEPTH = 0.1

NSEC = 4
SECSZ = HW // NSEC
WIN = 2048
NWIN = HW // WIN
CHUNK = 4096
LANES = 16
NCORES = 2
NSUB = 16


def _rnd(v):
    return v.astype(jnp.bfloat16).astype(jnp.float32)


def _cells_body(m_ref, k_ref, t_ref, depth_ref, cell_ref):
    b = pl.program_id(0)
    gx = _rnd(lax.broadcasted_iota(jnp.int32, (H, W), 0).astype(jnp.float32))
    gy = _rnd(lax.broadcasted_iota(jnp.int32, (H, W), 1).astype(jnp.float32))
    d = jnp.clip(depth_ref[0], MIN_DEPTH)
    t0 = _rnd(t_ref[b, 0] / d)
    t1 = _rnd(t_ref[b, 1] / d)
    t2 = _rnd(t_ref[b, 2] / d)

    def proj_row(i):
        x = (m_ref[b, i, 0] * gx + m_ref[b, i, 1] * gy) + m_ref[b, i, 2]
        y = (k_ref[i, 0] * t0 + k_ref[i, 1] * t1) + k_ref[i, 2] * t2
        ti = x + y
        return jnp.where(ti == 0.0, 0.0001, ti)

    n0 = proj_row(0)
    n1 = proj_row(1)
    n2 = proj_row(2)
    p0 = jnp.clip(n0 / n2, 0.0, float(H - 1)).astype(jnp.int32)
    p1 = jnp.clip(n1 / n2, 0.0, float(W - 1)).astype(jnp.int32)
    cell = p0 * W + p1
    nxt_col = jnp.roll(cell, -1, axis=1)
    nxt_row = jnp.roll(jnp.roll(cell, -1, axis=0), -1, axis=1)
    col = lax.broadcasted_iota(jnp.int32, (H, W), 1)
    row = lax.broadcasted_iota(jnp.int32, (H, W), 0)
    nxt = jnp.where(col < W - 1, nxt_col, nxt_row)
    keep = (cell != nxt) | ((row == H - 1) & (col == W - 1))
    cell_ref[0] = jnp.where(keep, cell, -1)


def _compute_cells(depth, M, K, T2):
    return pl.pallas_call(
        _cells_body,
        grid=(B,),
        in_specs=[
            pl.BlockSpec(memory_space=pltpu.SMEM),
            pl.BlockSpec(memory_space=pltpu.SMEM),
            pl.BlockSpec(memory_space=pltpu.SMEM),
            pl.BlockSpec((1, H, W), lambda b: (b, 0, 0)),
        ],
        out_specs=pl.BlockSpec((1, H, W), lambda b: (b, 0, 0)),
        out_shape=jax.ShapeDtypeStruct((B, H, W), jnp.int32),
    )(M, K, T2, depth)


def _winner_body(cell_hbm, winner_hbm, cellwin, winner_v, sidxbuf, svalbuf,
                 sem0, sem1):
    cid = lax.axis_index("c")
    sid = lax.axis_index("s")
    wid = sid * NCORES + cid
    batch = wid // 2
    half = wid % 2
    lane = lax.iota(jnp.int32, 16)
    nxt = jnp.minimum(lane + 1, 15)
    base0 = batch * HW
    sems = (sem0, sem1)

    for sp in range(2):
        sec = half * 2 + sp

        def zero_body(i, _):
            winner_v[pl.ds(i * 16, 16)] = jnp.zeros((16,), jnp.int32)
            return 0

        lax.fori_loop(0, SECSZ // 16, zero_body, 0)

        def process(bi, w):
            @plsc.parallel_loop(0, WIN // 16, 1, unroll=4)
            def _phase1(v):
                cells = cellwin[pl.ds(bi * WIN + v * 16, 16)]
                in_sec = lax.shift_right_logical(cells, 16) == sec
                local = cells & 0xFFFF
                key = jnp.where(in_sec, (local << 4) | lane,
                                jnp.int32(0x40000000) | lane)
                kp1 = (w * WIN + v * 16 + 1) + lane
                skey, sval = plsc.sort_key_val(key, kp1)
                sidx = lax.shift_right_logical(skey, 4)
                nkey = _take16(skey, nxt)
                is_last = (sidx != lax.shift_right_logical(nkey, 4)) | (lane == 15)
                m = is_last & (skey < (1 << 20))
                sidxbuf[pl.ds(v * 16, 16)] = jnp.where(m, sidx, -1)
                svalbuf[pl.ds(v * 16, 16)] = sval

            def scat_body(v, _):
                si = sidxbuf[pl.ds(v * 16, 16)]
                sv = svalbuf[pl.ds(v * 16, 16)]
                plsc.store_scatter(winner_v, [si], sv, mask=si >= 0)
                return 0

            lax.fori_loop(0, WIN // 16, scat_body, 0)

        def start(bi, w):
            pltpu.async_copy(cell_hbm.at[pl.ds(base0 + w * WIN, WIN)],
                             cellwin.at[pl.ds(bi * WIN, WIN)], sems[bi])

        def wait(bi, w):
            pltpu.make_async_copy(cell_hbm.at[pl.ds(base0 + w * WIN, WIN)],
                                  cellwin.at[pl.ds(bi * WIN, WIN)],
                                  sems[bi]).wait()

        start(0, jnp.int32(0))

        def pair_body(p, _):
            w0 = 2 * p
            start(1, w0 + 1)
            wait(0, w0)
            process(0, w0)

            @pl.when(p + 1 < NWIN // 2)
            def _():
                start(0, w0 + 2)

            wait(1, w0 + 1)
            process(1, w0 + 1)
            return 0

        lax.fori_loop(0, NWIN // 2, pair_body, 0)
        pltpu.sync_copy(winner_v,
                        winner_hbm.at[pl.ds(batch * HW + sec * SECSZ, SECSZ)])


def _winner_scatter(cells_flat):
    return pl.kernel(
        _winner_body,
        out_type=jax.ShapeDtypeStruct((B * HW,), jnp.int32),
        mesh=plsc.VectorSubcoreMesh(core_axis_name="c", subcore_axis_name="s",
                                    num_cores=NCORES, num_subcores=NSUB),
        scratch_types=[
            pltpu.VMEM((2 * WIN,), jnp.int32),
            pltpu.VMEM((SECSZ,), jnp.int32),
            pltpu.VMEM((WIN,), jnp.int32),
            pltpu.VMEM((WIN,), jnp.int32),
            pltpu.SemaphoreType.DMA,
            pltpu.SemaphoreType.DMA,
        ],
        compiler_params=pltpu.CompilerParams(needs_layout_passes=False),
    )(cells_flat)


def _value_body(winner_hbm, image_hbm, out_hbm, win_v, idxb, posb, valb, sem):
    cid = lax.axis_index("c")
    sid = lax.axis_index("s")
    wid = sid * NCORES + cid
    lane = lax.iota(jnp.int32, 16)
    swap = lane ^ 1

    def fire(rem):
        pltpu.async_copy(image_hbm.at[idxb.at[pl.ds(0, CHUNK)]], valb, sem).wait()

        def scat(j, _):
            vals = valb[pl.ds(j * 16, 16)]
            pos = posb[pl.ds(j * 16, 16)]
            msk = (j * 16 + lane) < rem
            plsc.store_scatter(win_v, [pos], plsc.bitcast(vals, jnp.int32),
                               mask=msk)
            return 0

        lax.fori_loop(0, CHUNK // 16, scat, 0)

    for tsk in range(2):
        t = wid * 2 + tsk
        batch = t // NSEC
        sec = t % NSEC
        for c in range(C):
            base_img = (batch * C + c) * HW

            pltpu.sync_copy(
                winner_hbm.at[pl.ds(batch * HW + sec * SECSZ, SECSZ)], win_v)

            def init_body(i, _):
                idxb[pl.ds(i * 16, 16)] = jnp.broadcast_to(base_img, (16,)).astype(jnp.int32)
                return 0

            lax.fori_loop(0, (CHUNK + 16) // 16, init_body, 0)

            def compact_body(v, off):
                wv = win_v[pl.ds(v * 16, 16)]
                m = wv > 0
                cnt = jnp.sum(m.astype(jnp.int32))

                def active(off):
                    plsc.store_compressed(idxb.at[pl.ds(off, 16)],
                                          (wv - 1) + base_img, mask=m)
                    plsc.store_compressed(posb.at[pl.ds(off, 16)],
                                          v * 16 + lane, mask=m)
                    off = off + cnt

                    def do_fire(off):
                        fire(jnp.int32(CHUNK))
                        idxb[pl.ds(0, 16)] = idxb[pl.ds(CHUNK, 16)]
                        posb[pl.ds(0, 16)] = posb[pl.ds(CHUNK, 16)]
                        return off - CHUNK

                    return lax.cond(off >= CHUNK, do_fire, lambda o: o, off)

                return lax.cond(cnt > 0, active, lambda o: o, off)

            off_fin = lax.fori_loop(0, SECSZ // 16, compact_body,
                                    jnp.int32(0))

            def tail_fire(off):
                fire(off)
                return 0

            lax.cond(off_fin > 0, tail_fire, lambda o: 0, off_fin)

            def pool_body(r, _):
                def grp_body(g, _):
                    a = plsc.bitcast(win_v[pl.ds((2 * r) * W + g * 16, 16)],
                                     jnp.float32)
                    b = plsc.bitcast(win_v[pl.ds((2 * r + 1) * W + g * 16, 16)],
                                     jnp.float32)
                    v = jnp.maximum(a, b)
                    sw = _take16(v, swap)
                    wz = plsc.bitcast(jnp.maximum(v, sw), jnp.int32)
                    win_v[pl.ds((2 * r) * W + g * 16, 16)] = wz
                    win_v[pl.ds((2 * r + 1) * W + g * 16, 16)] = wz
                    return 0

                lax.fori_loop(0, W // 16, grp_body, 0)
                return 0

            lax.fori_loop(0, SECSZ // (2 * W), pool_body, 0)

            out_off = ((batch * C + c) * H + sec * (H // NSEC)) * W
            pltpu.sync_copy(win_v, out_hbm.at[pl.ds(out_off, SECSZ)])


def _gather_pool(winner, image_flat):
    return pl.kernel(
        _value_body,
        out_type=jax.ShapeDtypeStruct((B * C * HW,), jnp.int32),
        mesh=plsc.VectorSubcoreMesh(core_axis_name="c", subcore_axis_name="s",
                                    num_cores=NCORES, num_subcores=NSUB),
        scratch_types=[
            pltpu.VMEM((SECSZ,), jnp.int32),
            pltpu.VMEM((CHUNK + 16,), jnp.int32),
            pltpu.VMEM((CHUNK + 16,), jnp.int32),
            pltpu.VMEM((CHUNK,), jnp.float32),
            pltpu.SemaphoreType.DMA,
        ],
        compiler_params=pltpu.CompilerParams(needs_layout_passes=False),
    )(winner, image_flat)


def kernel(image, depth, T, R, K, Kinv):
    M = jnp.matmul(jnp.matmul(K, R), Kinv)
    Mr = M.astype(jnp.bfloat16).astype(jnp.float32)
    Kr = K.astype(jnp.bfloat16).astype(jnp.float32)
    T2 = T[:, :, 0]
    cells = _compute_cells(depth, Mr, Kr, T2)
    winner = _winner_scatter(cells.reshape(B * HW))
    out_i = _gather_pool(winner, image.reshape(B * C * HW))
    return lax.bitcast_convert_type(out_i, jnp.float32).reshape(B, C, H, W)


# speedup vs baseline: 57.1697x; 1.4269x over previous
import jax
import jax.numpy as jnp
from jax import lax
from jax.experimental import pallas as pl
from jax.experimental.pallas import tpu as pltpu
from jax.experimental.pallas import tpu_sc as plsc

def _take16(x, idx):
    dnums = lax.GatherDimensionNumbers(
        offset_dims=(), collapsed_slice_dims=(0,), start_index_map=(0,))
    return lax.gather(x, idx[:, None], dnums, slice_sizes=(1,),
                      mode=lax.GatherScatterMode.PROMISE_IN_BOUNDS)


B, C, H, W = 16, 3, 512, 512
HW = H * W
MIN_DEPTH = 0.1

NSEC = 4
SECSZ = HW // NSEC
WIN = 2048
NWIN = HW // WIN
CHUNK = 4096
SLACK = 128
LANES = 16
NCORES = 2
NSUB = 16


def _rnd(v):
    return v.astype(jnp.bfloat16).astype(jnp.float32)


def _cells_body(m_ref, k_ref, t_ref, depth_ref, cell_ref):
    b = pl.program_id(0)
    gx = _rnd(lax.broadcasted_iota(jnp.int32, (H, W), 0).astype(jnp.float32))
    gy = _rnd(lax.broadcasted_iota(jnp.int32, (H, W), 1).astype(jnp.float32))
    d = jnp.clip(depth_ref[0], MIN_DEPTH)
    t0 = _rnd(t_ref[b, 0] / d)
    t1 = _rnd(t_ref[b, 1] / d)
    t2 = _rnd(t_ref[b, 2] / d)

    def proj_row(i):
        x = (m_ref[b, i, 0] * gx + m_ref[b, i, 1] * gy) + m_ref[b, i, 2]
        y = (k_ref[i, 0] * t0 + k_ref[i, 1] * t1) + k_ref[i, 2] * t2
        ti = x + y
        return jnp.where(ti == 0.0, 0.0001, ti)

    n0 = proj_row(0)
    n1 = proj_row(1)
    n2 = proj_row(2)
    p0 = jnp.clip(n0 / n2, 0.0, float(H - 1)).astype(jnp.int32)
    p1 = jnp.clip(n1 / n2, 0.0, float(W - 1)).astype(jnp.int32)
    cell = p0 * W + p1
    nxt_col = jnp.roll(cell, -1, axis=1)
    nxt_row = jnp.roll(jnp.roll(cell, -1, axis=0), -1, axis=1)
    col = lax.broadcasted_iota(jnp.int32, (H, W), 1)
    row = lax.broadcasted_iota(jnp.int32, (H, W), 0)
    nxt = jnp.where(col < W - 1, nxt_col, nxt_row)
    keep = (cell != nxt) | ((row == H - 1) & (col == W - 1))
    cell_ref[0] = jnp.where(keep, cell, -1)


def _compute_cells(depth, M, K, T2):
    return pl.pallas_call(
        _cells_body,
        grid=(B,),
        in_specs=[
            pl.BlockSpec(memory_space=pltpu.SMEM),
            pl.BlockSpec(memory_space=pltpu.SMEM),
            pl.BlockSpec(memory_space=pltpu.SMEM),
            pl.BlockSpec((1, H, W), lambda b: (b, 0, 0)),
        ],
        out_specs=pl.BlockSpec((1, H, W), lambda b: (b, 0, 0)),
        out_shape=jax.ShapeDtypeStruct((B, H, W), jnp.int32),
    )(M, K, T2, depth)


def _winner_body(cell_hbm, winner_hbm, cellwin, winner_v, sidxbuf, svalbuf,
                 sem0, sem1):
    cid = lax.axis_index("c")
    sid = lax.axis_index("s")
    wid = sid * NCORES + cid
    batch = wid // 2
    half = wid % 2
    lane = lax.iota(jnp.int32, 16)
    nxt = jnp.minimum(lane + 1, 15)
    base0 = batch * HW
    sems = (sem0, sem1)

    for sp in range(2):
        sec = half * 2 + sp

        def zero_body(i, _):
            winner_v[pl.ds(i * 16, 16)] = jnp.zeros((16,), jnp.int32)
            return 0

        lax.fori_loop(0, SECSZ // 16, zero_body, 0)

        def process(bi, w):
            @plsc.parallel_loop(0, WIN // 16, 1, unroll=4)
            def _phase1(v):
                cells = cellwin[pl.ds(bi * WIN + v * 16, 16)]
                in_sec = lax.shift_right_logical(cells, 16) == sec
                local = cells & 0xFFFF
                key = jnp.where(in_sec, (local << 4) | lane,
                                jnp.int32(0x40000000) | lane)
                kp1 = (w * WIN + v * 16 + 1) + lane
                skey, sval = plsc.sort_key_val(key, kp1)
                sidx = lax.shift_right_logical(skey, 4)
                nkey = _take16(skey, nxt)
                is_last = (sidx != lax.shift_right_logical(nkey, 4)) | (lane == 15)
                m = is_last & (skey < (1 << 20))
                sidxbuf[pl.ds(v * 16, 16)] = jnp.where(m, sidx, -1)
                svalbuf[pl.ds(v * 16, 16)] = sval

            def scat_body(v, _):
                si = sidxbuf[pl.ds(v * 16, 16)]
                sv = svalbuf[pl.ds(v * 16, 16)]
                plsc.store_scatter(winner_v, [si], sv, mask=si >= 0)
                return 0

            lax.fori_loop(0, WIN // 16, scat_body, 0)

        def start(bi, w):
            pltpu.async_copy(cell_hbm.at[pl.ds(base0 + w * WIN, WIN)],
                             cellwin.at[pl.ds(bi * WIN, WIN)], sems[bi])

        def wait(bi, w):
            pltpu.make_async_copy(cell_hbm.at[pl.ds(base0 + w * WIN, WIN)],
                                  cellwin.at[pl.ds(bi * WIN, WIN)],
                                  sems[bi]).wait()

        start(0, jnp.int32(0))

        def pair_body(p, _):
            w0 = 2 * p
            start(1, w0 + 1)
            wait(0, w0)
            process(0, w0)

            @pl.when(p + 1 < NWIN // 2)
            def _():
                start(0, w0 + 2)

            wait(1, w0 + 1)
            process(1, w0 + 1)
            return 0

        lax.fori_loop(0, NWIN // 2, pair_body, 0)
        pltpu.sync_copy(winner_v,
                        winner_hbm.at[pl.ds(batch * HW + sec * SECSZ, SECSZ)])


def _winner_scatter(cells_flat):
    return pl.kernel(
        _winner_body,
        out_type=jax.ShapeDtypeStruct((B * HW,), jnp.int32),
        mesh=plsc.VectorSubcoreMesh(core_axis_name="c", subcore_axis_name="s",
                                    num_cores=NCORES, num_subcores=NSUB),
        scratch_types=[
            pltpu.VMEM((2 * WIN,), jnp.int32),
            pltpu.VMEM((SECSZ,), jnp.int32),
            pltpu.VMEM((WIN,), jnp.int32),
            pltpu.VMEM((WIN,), jnp.int32),
            pltpu.SemaphoreType.DMA,
            pltpu.SemaphoreType.DMA,
        ],
        compiler_params=pltpu.CompilerParams(needs_layout_passes=False),
    )(cells_flat)


def _value_body(winner_hbm, image_hbm, out_hbm, win_v, idxb, posb, valb, sem):
    cid = lax.axis_index("c")
    sid = lax.axis_index("s")
    wid = sid * NCORES + cid
    lane = lax.iota(jnp.int32, 16)
    swap = lane ^ 1
    fifteen = jnp.minimum(lane + 15, 15)

    def fire(rem):
        pltpu.async_copy(image_hbm.at[idxb.at[pl.ds(0, CHUNK)]], valb, sem).wait()

        def scat(j, _):
            vals = valb[pl.ds(j * 16, 16)]
            pos = posb[pl.ds(j * 16, 16)]
            msk = (j * 16 + lane) < rem
            plsc.store_scatter(win_v, [pos], plsc.bitcast(vals, jnp.int32),
                               mask=msk)
            return 0

        lax.fori_loop(0, CHUNK // 16, scat, 0)

    for tsk in range(2):
        t = wid * 2 + tsk
        batch = t // NSEC
        sec = t % NSEC
        for c in range(C):
            base_img = (batch * C + c) * HW

            pltpu.sync_copy(
                winner_hbm.at[pl.ds(batch * HW + sec * SECSZ, SECSZ)], win_v)

            def init_body(i, _):
                idxb[pl.ds(i * 16, 16)] = jnp.broadcast_to(base_img, (16,)).astype(jnp.int32)
                return 0

            lax.fori_loop(0, (CHUNK + SLACK) // 16, init_body, 0)

            def grp_body(g, off_vec):
                for j in range(8):
                    v = g * 8 + j
                    wv = win_v[pl.ds(v * 16, 16)]
                    m = wv > 0
                    pos = plsc.cumsum(m.astype(jnp.int32))
                    total = _take16(pos, fifteen)
                    tgt = (off_vec + pos) - 1
                    plsc.store_scatter(idxb, [tgt], (wv - 1) + base_img, mask=m)
                    plsc.store_scatter(posb, [tgt], v * 16 + lane, mask=m)
                    off_vec = off_vec + total
                off_s = jnp.max(off_vec)

                def do_fire(ov):
                    fire(jnp.int32(CHUNK))
                    for j in range(SLACK // 16):
                        idxb[pl.ds(j * 16, 16)] = idxb[pl.ds(CHUNK + j * 16, 16)]
                        posb[pl.ds(j * 16, 16)] = posb[pl.ds(CHUNK + j * 16, 16)]
                    return ov - CHUNK

                return lax.cond(off_s >= CHUNK, do_fire, lambda ov: ov, off_vec)

            off_vec = lax.fori_loop(0, SECSZ // 16 // 8, grp_body,
                                    jnp.zeros((16,), jnp.int32))
            off_fin = jnp.max(off_vec)

            def tail_fire(off):
                fire(off)
                return 0

            lax.cond(off_fin > 0, tail_fire, lambda o: 0, off_fin)

            @plsc.parallel_loop(0, SECSZ // (2 * W), 1, unroll=2)
            def _pool(r):
                for g in range(W // 16):
                    a = plsc.bitcast(win_v[pl.ds((2 * r) * W + g * 16, 16)],
                                     jnp.float32)
                    b = plsc.bitcast(win_v[pl.ds((2 * r + 1) * W + g * 16, 16)],
                                     jnp.float32)
                    v = jnp.maximum(a, b)
                    sw = _take16(v, swap)
                    wz = plsc.bitcast(jnp.maximum(v, sw), jnp.int32)
                    win_v[pl.ds((2 * r) * W + g * 16, 16)] = wz
                    win_v[pl.ds((2 * r + 1) * W + g * 16, 16)] = wz

            out_off = ((batch * C + c) * H + sec * (H // NSEC)) * W
            pltpu.sync_copy(win_v, out_hbm.at[pl.ds(out_off, SECSZ)])


def _gather_pool(winner, image_flat):
    return pl.kernel(
        _value_body,
        out_type=jax.ShapeDtypeStruct((B * C * HW,), jnp.int32),
        mesh=plsc.VectorSubcoreMesh(core_axis_name="c", subcore_axis_name="s",
                                    num_cores=NCORES, num_subcores=NSUB),
        scratch_types=[
            pltpu.VMEM((SECSZ,), jnp.int32),
            pltpu.VMEM((CHUNK + SLACK,), jnp.int32),
            pltpu.VMEM((CHUNK + SLACK,), jnp.int32),
            pltpu.VMEM((CHUNK,), jnp.float32),
            pltpu.SemaphoreType.DMA,
        ],
        compiler_params=pltpu.CompilerParams(needs_layout_passes=False),
    )(winner, image_flat)


def kernel(image, depth, T, R, K, Kinv):
    M = jnp.matmul(jnp.matmul(K, R), Kinv)
    Mr = M.astype(jnp.bfloat16).astype(jnp.float32)
    Kr = K.astype(jnp.bfloat16).astype(jnp.float32)
    T2 = T[:, :, 0]
    cells = _compute_cells(depth, Mr, Kr, T2)
    winner = _winner_scatter(cells.reshape(B * HW))
    out_i = _gather_pool(winner, image.reshape(B * C * HW))
    return lax.bitcast_convert_type(out_i, jnp.float32).reshape(B, C, H, W)
